# baseline (speedup 1.0000x reference)
import functools

import jax
import jax.numpy as jnp
from jax import lax
from jax.experimental import pallas as pl
from jax.experimental.pallas import tpu as pltpu
from jax.experimental.pallas import tpu_sc as plsc

_B, _T, _H, _W = 3, 300, 256, 256
_N = 32
_CROWS = 32
_CPF = _H // _CROWS
_NBUF = 12
_PRO = 8
_NCHUNK = _CPF * _B


def _sc_subsample(x):
    mesh = plsc.VectorSubcoreMesh(core_axis_name="c", subcore_axis_name="s")

    @functools.partial(
        pl.kernel,
        mesh=mesh,
        out_type=jax.ShapeDtypeStruct((_B, _N, _H, _W), jnp.float32),
        scratch_types=[
            pltpu.VMEM((_NBUF, _CROWS, _W), jnp.float32),
            pltpu.SemaphoreType.DMA,
            pltpu.SemaphoreType.DMA,
        ],
    )
    def k(x_hbm, out_hbm, buf, sem_in, sem_out):
        c = lax.axis_index("c")
        s = lax.axis_index("s")
        w = s * 2 + c
        src = lax.div(w * (_T - 1), _N - 1)

        def make_in(u):
            b, h = divmod(u, _CPF)
            return pltpu.make_async_copy(
                x_hbm.at[b, src, pl.ds(h * _CROWS, _CROWS)],
                buf.at[u % _NBUF],
                sem_in,
            )

        def make_out(u):
            b, h = divmod(u, _CPF)
            return pltpu.make_async_copy(
                buf.at[u % _NBUF],
                out_hbm.at[b, w, pl.ds(h * _CROWS, _CROWS)],
                sem_out,
            )

        ins = [make_in(u) for u in range(_NCHUNK)]
        outs = [make_out(u) for u in range(_NCHUNK)]
        out_waited = [False] * _NCHUNK

        for u in range(_PRO):
            ins[u].start()
        for t in range(_NCHUNK):
            ins[t].wait()
            outs[t].start()
            u = t + _PRO
            if u < _NCHUNK:
                if u - _NBUF >= 0:
                    outs[u - _NBUF].wait()
                    out_waited[u - _NBUF] = True
                ins[u].start()
        for t in range(_NCHUNK):
            if not out_waited[t]:
                outs[t].wait()

    return k(x)


def kernel(x):
    return _sc_subsample(x)


# speedup vs baseline: 1.0365x; 1.0365x over previous
import functools

import jax
import jax.numpy as jnp
from jax import lax
from jax.experimental import pallas as pl
from jax.experimental.pallas import tpu as pltpu
from jax.experimental.pallas import tpu_sc as plsc

_B, _T, _H, _W = 3, 300, 256, 256
_N = 32
_CROWS = 128
_CPF = _H // _CROWS
_NBUF = 3
_PRO = 2
_NCHUNK = _CPF * _B


def _sc_subsample(x):
    mesh = plsc.VectorSubcoreMesh(core_axis_name="c", subcore_axis_name="s")

    @functools.partial(
        pl.kernel,
        mesh=mesh,
        out_type=jax.ShapeDtypeStruct((_B, _N, _H, _W), jnp.float32),
        scratch_types=[
            pltpu.VMEM((_NBUF, _CROWS, _W), jnp.float32),
            pltpu.SemaphoreType.DMA,
            pltpu.SemaphoreType.DMA,
        ],
    )
    def k(x_hbm, out_hbm, buf, sem_in, sem_out):
        c = lax.axis_index("c")
        s = lax.axis_index("s")
        w = s * 2 + c
        src = lax.div(w * (_T - 1), _N - 1)

        def make_in(u):
            b, h = divmod(u, _CPF)
            return pltpu.make_async_copy(
                x_hbm.at[b, src, pl.ds(h * _CROWS, _CROWS)],
                buf.at[u % _NBUF],
                sem_in,
            )

        def make_out(u):
            b, h = divmod(u, _CPF)
            return pltpu.make_async_copy(
                buf.at[u % _NBUF],
                out_hbm.at[b, w, pl.ds(h * _CROWS, _CROWS)],
                sem_out,
            )

        ins = [make_in(u) for u in range(_NCHUNK)]
        outs = [make_out(u) for u in range(_NCHUNK)]
        out_waited = [False] * _NCHUNK

        for u in range(_PRO):
            ins[u].start()
        for t in range(_NCHUNK):
            ins[t].wait()
            outs[t].start()
            u = t + _PRO
            if u < _NCHUNK:
                if u - _NBUF >= 0:
                    outs[u - _NBUF].wait()
                    out_waited[u - _NBUF] = True
                ins[u].start()
        for t in range(_NCHUNK):
            if not out_waited[t]:
                outs[t].wait()

    return k(x)


def kernel(x):
    return _sc_subsample(x)


# speedup vs baseline: 1.0373x; 1.0007x over previous
import functools

import jax
import jax.numpy as jnp
from jax import lax
from jax.experimental import pallas as pl
from jax.experimental.pallas import tpu as pltpu
from jax.experimental.pallas import tpu_sc as plsc

_B, _T, _H, _W = 3, 300, 256, 256
_N = 32
_CROWS = 128
_NBUF = 3
_PRO = 2

_CHUNKS = [
    (0, 0, 64), (0, 64, 64), (0, 128, 128),
    (1, 0, 128), (1, 128, 128),
    (2, 0, 128), (2, 128, 64), (2, 192, 64),
]
_NCHUNK = len(_CHUNKS)


def _sc_subsample(x):
    mesh = plsc.VectorSubcoreMesh(core_axis_name="c", subcore_axis_name="s")

    @functools.partial(
        pl.kernel,
        mesh=mesh,
        out_type=jax.ShapeDtypeStruct((_B, _N, _H, _W), jnp.float32),
        scratch_types=[
            pltpu.VMEM((_NBUF, _CROWS, _W), jnp.float32),
            pltpu.SemaphoreType.DMA,
            pltpu.SemaphoreType.DMA,
        ],
    )
    def k(x_hbm, out_hbm, buf, sem_in, sem_out):
        c = lax.axis_index("c")
        s = lax.axis_index("s")
        w = s * 2 + c
        src = lax.div(w * (_T - 1), _N - 1)

        def make_in(u):
            b, r, nr = _CHUNKS[u]
            return pltpu.make_async_copy(
                x_hbm.at[b, src, pl.ds(r, nr)],
                buf.at[u % _NBUF, pl.ds(0, nr)],
                sem_in,
            )

        def make_out(u):
            b, r, nr = _CHUNKS[u]
            return pltpu.make_async_copy(
                buf.at[u % _NBUF, pl.ds(0, nr)],
                out_hbm.at[b, w, pl.ds(r, nr)],
                sem_out,
            )

        ins = [make_in(u) for u in range(_NCHUNK)]
        outs = [make_out(u) for u in range(_NCHUNK)]
        out_waited = [False] * _NCHUNK

        for u in range(_PRO):
            ins[u].start()
        for t in range(_NCHUNK):
            ins[t].wait()
            outs[t].start()
            u = t + _PRO
            if u < _NCHUNK:
                if u - _NBUF >= 0:
                    outs[u - _NBUF].wait()
                    out_waited[u - _NBUF] = True
                ins[u].start()
        for t in range(_NCHUNK):
            if not out_waited[t]:
                outs[t].wait()

    return k(x)


def kernel(x):
    return _sc_subsample(x)
